# baseline (speedup 1.0000x reference)
import functools

import jax
import jax.numpy as jnp
from jax import lax
from jax.experimental import pallas as pl
from jax.experimental.pallas import tpu as pltpu
from jax.experimental.pallas import tpu_sc as plsc

T = 2048
H = 2048
NH = 16
NKV = 4
HD = 128
E = 8
TOPK = 2
DFF = 768
EPS = 1e-06
THETA = 10000.0

BM = 256
S = T * TOPK
GBM = 256
GSH = 8
NT = S // GBM
G = NT + E - 1
PREC = jax.lax.Precision.DEFAULT


def _rms_rows(x, w):
    return x * jax.lax.rsqrt(jnp.mean(x * x, axis=-1, keepdims=True) + EPS) * w


def _qkv_kernel(x_ref, ln1_ref, w_ref, qn_ref, kn_ref, pos_ref,
                q_ref, k_ref, v_ref):
    x = x_ref[...]
    h = _rms_rows(x, ln1_ref[...])
    qkv = jnp.dot(h, w_ref[...], preferred_element_type=jnp.float32,
                  precision=PREC)

    pos = pos_ref[:, 0].astype(jnp.float32)
    half = HD // 2
    inv = 1.0 / (THETA ** (jnp.arange(0, half, dtype=jnp.int32).astype(jnp.float32) / half))
    f = pos[:, None] * inv[None, :]
    cos = jnp.cos(f)
    sin = jnp.sin(f)

    def norm_rope(s, nw):
        s = _rms_rows(s, nw)
        x1 = s[:, :half]
        x2 = s[:, half:]
        return jnp.concatenate([x1 * cos - x2 * sin, x2 * cos + x1 * sin],
                               axis=-1)

    qn = qn_ref[...]
    kn = kn_ref[...]
    for hidx in range(NH):
        sl = slice(hidx * HD, (hidx + 1) * HD)
        q_ref[:, sl] = norm_rope(qkv[:, sl], qn)
    for hidx in range(NKV):
        sl = slice(NH * HD + hidx * HD, NH * HD + (hidx + 1) * HD)
        osl = slice(hidx * HD, (hidx + 1) * HD)
        k_ref[:, osl] = norm_rope(qkv[:, sl], kn)
        vsl = slice((NH + NKV) * HD + hidx * HD, (NH + NKV) * HD + (hidx + 1) * HD)
        v_ref[:, osl] = qkv[:, vsl]


def _attn_kernel(qi_r, kj_r, q_ref, k_ref, v_ref, o_ref,
                 acc_ref, m_ref, l_ref):
    sx = pl.program_id(1)
    qi = qi_r[sx]
    kj = kj_r[sx]
    q = q_ref[...]
    k = k_ref[...]
    v = v_ref[...]
    s = jax.lax.dot_general(q, k, (((1,), (1,)), ((), ())),
                            preferred_element_type=jnp.float32,
                            precision=PREC) * (HD ** -0.5)
    row = qi * BM + jax.lax.broadcasted_iota(jnp.int32, s.shape, 0)
    col = kj * BM + jax.lax.broadcasted_iota(jnp.int32, s.shape, 1)
    s = jnp.where(col <= row, s, -1e30)

    first = kj == 0
    m_old = jnp.where(first, -jnp.inf, m_ref[:, :1])
    l_old = jnp.where(first, 0.0, l_ref[:, :1])
    acc_old = jnp.where(first, 0.0, acc_ref[...])

    mc = jnp.max(s, axis=-1, keepdims=True)
    mn = jnp.maximum(m_old, mc)
    p = jnp.exp(s - mn)
    corr = jnp.exp(m_old - mn)
    l_new = l_old * corr + jnp.sum(p, axis=-1, keepdims=True)
    acc_new = acc_old * corr + jnp.dot(
        p, v, preferred_element_type=jnp.float32, precision=PREC)
    m_ref[:, :1] = mn
    l_ref[:, :1] = l_new
    acc_ref[...] = acc_new

    @pl.when(kj == qi)
    def _():
        o_ref[...] = acc_new / l_new


def _oproj_kernel(o_ref, w_ref, res_ref, ln2_ref, wr_ref,
                  h_ref, h2_ref, lg_ref):
    h = res_ref[...] + jnp.dot(
        o_ref[...], w_ref[...], preferred_element_type=jnp.float32,
        precision=PREC)
    h_ref[...] = h
    h2 = _rms_rows(h, ln2_ref[...])
    h2_ref[...] = h2
    lg_ref[...] = jnp.dot(h2, wr_ref[...], preferred_element_type=jnp.float32,
                          precision=PREC)


def _cumsum_rows(x):
    s = x
    shift = 1
    while shift < x.shape[0]:
        z = jnp.zeros((shift, x.shape[1]), x.dtype)
        s = s + jnp.concatenate([z, s[:-shift]], axis=0)
        shift *= 2
    return s


def _cumsum_lanes_excl(x):
    n = x.shape[1]
    s = jnp.zeros_like(x)
    acc = x
    shift = 1
    while shift < n:
        z = jnp.zeros((x.shape[0], shift), x.dtype)
        acc = acc + jnp.concatenate([z, acc[:, :-shift]], axis=1)
        shift *= 2
    s = acc - x
    return s


def _router_kernel(lg_ref,
                   p1_ref, p2_ref, w1_ref, w2_ref,
                   mt_ref, gid_ref, lo_ref, hi_ref):
    logits = lg_ref[...]
    mx = jnp.max(logits, axis=-1, keepdims=True)
    ex = jnp.exp(logits - mx)
    p = ex / jnp.sum(ex, axis=-1, keepdims=True)
    lane = jax.lax.broadcasted_iota(jnp.int32, p.shape, 1)
    m1 = jnp.max(p, axis=-1, keepdims=True)
    i1 = jnp.min(jnp.where(p == m1, lane, E), axis=-1, keepdims=True)
    oh1 = lane == i1
    pm = jnp.where(oh1, -jnp.inf, p)
    m2 = jnp.max(pm, axis=-1, keepdims=True)
    i2 = jnp.min(jnp.where(pm == m2, lane, E), axis=-1, keepdims=True)
    oh2 = lane == i2
    tot = m1 + m2
    w1_ref[...] = m1 / tot
    w2_ref[...] = m2 / tot

    cnt = oh1.astype(jnp.int32) + oh2.astype(jnp.int32)
    incl = _cumsum_rows(cnt)
    excl = incl - cnt
    sizes = incl[T - 1:T, :]
    offs = _cumsum_lanes_excl(sizes)
    pos = offs + excl
    p1_ref[...] = jnp.sum(jnp.where(oh1, pos, 0), axis=-1, keepdims=True)
    p2_ref[...] = jnp.sum(jnp.where(oh2, pos, 0), axis=-1, keepdims=True)

    off_lo = offs
    off_hi = offs + sizes
    start_t = off_lo >> GSH
    end_t = (off_hi - 1) >> GSH
    nt = jnp.where(sizes > 0, end_t - start_t + 1, 0)
    estart = _cumsum_lanes_excl(nt)
    n_entries = estart[:, E - 1:E] + nt[:, E - 1:E]

    gi = jax.lax.broadcasted_iota(jnp.int32, (E, G), 1)
    erow = jax.lax.broadcasted_iota(jnp.int32, (E, G), 0)
    est_b = jnp.broadcast_to(estart.reshape(E, 1), (E, G))
    nt_b = jnp.broadcast_to(nt.reshape(E, 1), (E, G))
    st_b = jnp.broadcast_to(start_t.reshape(E, 1), (E, G))
    lo_b = jnp.broadcast_to(off_lo.reshape(E, 1), (E, G))
    hi_b = jnp.broadcast_to(off_hi.reshape(E, 1), (E, G))
    ind = (gi >= est_b) & (gi < est_b + nt_b)
    iz = jnp.zeros((E, G), jnp.int32)
    mt = jnp.sum(jnp.where(ind, st_b + gi - est_b, iz), axis=0, keepdims=True)
    gid = jnp.sum(jnp.where(ind, erow, iz), axis=0, keepdims=True)
    elo = jnp.sum(jnp.where(ind, lo_b, iz), axis=0, keepdims=True)
    ehi = jnp.sum(jnp.where(ind, hi_b, iz), axis=0, keepdims=True)
    tile_lo = mt << GSH
    tile_hi = tile_lo + GBM
    lo = jnp.maximum(elo, tile_lo)
    hi = jnp.minimum(ehi, tile_hi)
    gpad = jax.lax.broadcasted_iota(jnp.int32, (1, G), 1) >= n_entries
    mt_ref[...] = jnp.where(gpad, NT - 1, mt)
    gid_ref[...] = jnp.where(gpad, 0, gid)
    lo_ref[...] = jnp.where(gpad, 0, lo)
    hi_ref[...] = jnp.where(gpad, 0, hi)


def _gmm_kernel(mt_ref, gid_ref, lo_ref, hi_ref,
                x_ref, wg_ref, wu_ref, wd_ref, y_ref):
    g = pl.program_id(0)
    x = x_ref[...]
    gt = jnp.dot(x, wg_ref[0], preferred_element_type=jnp.float32,
                 precision=PREC)
    up = jnp.dot(x, wu_ref[0], preferred_element_type=jnp.float32,
                 precision=PREC)
    a = (gt * jax.lax.logistic(gt)) * up
    y = jnp.dot(a, wd_ref[0], preferred_element_type=jnp.float32,
                precision=PREC)
    rows = mt_ref[g] * GBM + jax.lax.broadcasted_iota(
        jnp.int32, (GBM, 1), 0)
    mask = (rows >= lo_ref[g]) & (rows < hi_ref[g])
    contrib = jnp.where(mask, y, 0.0)

    prev = jnp.where(g == 0, -1, mt_ref[jnp.maximum(g - 1, 0)])
    first = mt_ref[g] != prev

    @pl.when(first)
    def _():
        y_ref[...] = contrib

    @pl.when(jnp.logical_not(first))
    def _():
        y_ref[...] += contrib


NWORK = 32
TPW = T // NWORK
CH = 16


@functools.cache
def _build_sc_dispatch():
    mesh = plsc.VectorSubcoreMesh(core_axis_name="c", subcore_axis_name="s")

    @functools.partial(
        pl.kernel, mesh=mesh,
        out_type=jax.ShapeDtypeStruct((S, H), jnp.float32),
        scratch_types=[
            pltpu.VMEM((CH,), jnp.int32),
            pltpu.VMEM((CH,), jnp.int32),
            pltpu.VMEM((CH, H), jnp.float32),
            pltpu.SemaphoreType.DMA,
        ],
    )
    def dispatch(h2_hbm, p1_hbm, p2_hbm, x_hbm, idx1_v, idx2_v, rows_v, sem):
        wid = lax.axis_index("s") * 2 + lax.axis_index("c")
        for j in range(TPW // CH):
            base = wid * TPW + j * CH
            pltpu.sync_copy(h2_hbm.at[pl.ds(base, CH)], rows_v)
            pltpu.sync_copy(p1_hbm.at[pl.ds(base, CH)], idx1_v)
            pltpu.sync_copy(p2_hbm.at[pl.ds(base, CH)], idx2_v)
            pltpu.async_copy(rows_v, x_hbm.at[idx1_v], sem).wait()
            pltpu.async_copy(rows_v, x_hbm.at[idx2_v], sem).wait()

    return dispatch


@functools.cache
def _build_sc_combine():
    mesh = plsc.VectorSubcoreMesh(core_axis_name="c", subcore_axis_name="s")

    @functools.partial(
        pl.kernel, mesh=mesh,
        out_type=jax.ShapeDtypeStruct((T, H), jnp.float32),
        scratch_types=[
            pltpu.VMEM((CH,), jnp.int32),
            pltpu.VMEM((CH,), jnp.int32),
            pltpu.VMEM((CH,), jnp.float32),
            pltpu.VMEM((CH,), jnp.float32),
            pltpu.VMEM((CH, H), jnp.float32),
            pltpu.VMEM((CH, H), jnp.float32),
            pltpu.VMEM((CH, H), jnp.float32),
            pltpu.SemaphoreType.DMA,
        ],
    )
    def combine(ys_hbm, res_hbm, p1_hbm, p2_hbm, w1_hbm, w2_hbm, out_hbm,
                idx1_v, idx2_v, w1_v, w2_v, y1_v, y2_v, acc_v, sem):
        wid = lax.axis_index("s") * 2 + lax.axis_index("c")
        for j in range(TPW // CH):
            base = wid * TPW + j * CH
            pltpu.sync_copy(p1_hbm.at[pl.ds(base, CH)], idx1_v)
            pltpu.sync_copy(p2_hbm.at[pl.ds(base, CH)], idx2_v)
            pltpu.sync_copy(w1_hbm.at[pl.ds(base, CH)], w1_v)
            pltpu.sync_copy(w2_hbm.at[pl.ds(base, CH)], w2_v)
            pltpu.sync_copy(res_hbm.at[pl.ds(base, CH)], acc_v)
            pltpu.async_copy(ys_hbm.at[idx1_v], y1_v, sem).wait()
            pltpu.async_copy(ys_hbm.at[idx2_v], y2_v, sem).wait()
            w1x = w1_v[...]
            w2x = w2_v[...]
            for t in range(CH):
                tvec = jnp.full((16,), t, jnp.int32)
                w1b = w1x.at[tvec].get(mode="promise_in_bounds")
                w2b = w2x.at[tvec].get(mode="promise_in_bounds")

                def body(c, _, t=t, w1b=w1b, w2b=w2b):
                    sl = pl.ds(c * 16, 16)
                    acc_v[t, sl] = (acc_v[t, sl] + w1b * y1_v[t, sl]
                                    + w2b * y2_v[t, sl])
                    return 0

                lax.fori_loop(0, H // 16, body, 0, unroll=8)
            pltpu.sync_copy(acc_v, out_hbm.at[pl.ds(base, CH)])

    return combine


def _sc_dispatch(h2, p1f, p2f):
    return _build_sc_dispatch()(h2, p1f, p2f)


def _sc_combine(ys, res, p1f, p2f, w1f, w2f):
    return _build_sc_combine()(ys, res, p1f, p2f, w1f, w2f)


@jax.jit
def kernel(hidden_states, positions, ln1_w, w_qkv, q_norm_w, k_norm_w, w_o,
           ln2_w, w_router, w_gate, w_up, w_down):
    posc = positions.reshape(T, 1)
    ln1 = ln1_w.reshape(1, H)
    ln2 = ln2_w.reshape(1, H)
    qn = q_norm_w.reshape(1, HD)
    kn = k_norm_w.reshape(1, HD)
    nm = T // BM

    q, k, v = pl.pallas_call(
        _qkv_kernel,
        grid=(nm,),
        in_specs=[
            pl.BlockSpec((BM, H), lambda i: (i, 0)),
            pl.BlockSpec((1, H), lambda i: (0, 0)),
            pl.BlockSpec((H, (NH + 2 * NKV) * HD), lambda i: (0, 0)),
            pl.BlockSpec((1, HD), lambda i: (0, 0)),
            pl.BlockSpec((1, HD), lambda i: (0, 0)),
            pl.BlockSpec((BM, 1), lambda i: (i, 0)),
        ],
        out_specs=[
            pl.BlockSpec((BM, NH * HD), lambda i: (i, 0)),
            pl.BlockSpec((BM, NKV * HD), lambda i: (i, 0)),
            pl.BlockSpec((BM, NKV * HD), lambda i: (i, 0)),
        ],
        out_shape=[
            jax.ShapeDtypeStruct((T, NH * HD), jnp.float32),
            jax.ShapeDtypeStruct((T, NKV * HD), jnp.float32),
            jax.ShapeDtypeStruct((T, NKV * HD), jnp.float32),
        ],
    )(hidden_states, ln1, w_qkv, qn, kn, posc)

    rep = NH // NKV
    pairs = [(i, j) for i in range(nm) for j in range(i + 1)]
    qi_arr = jnp.array([p[0] for p in pairs], jnp.int32)
    kj_arr = jnp.array([p[1] for p in pairs], jnp.int32)
    o = pl.pallas_call(
        _attn_kernel,
        grid_spec=pltpu.PrefetchScalarGridSpec(
            num_scalar_prefetch=2,
            grid=(NH, len(pairs)),
            in_specs=[
                pl.BlockSpec((BM, HD), lambda h, s, qi, kj: (qi[s], h)),
                pl.BlockSpec((BM, HD), lambda h, s, qi, kj: (kj[s], h // rep)),
                pl.BlockSpec((BM, HD), lambda h, s, qi, kj: (kj[s], h // rep)),
            ],
            out_specs=pl.BlockSpec((BM, HD), lambda h, s, qi, kj: (qi[s], h)),
            scratch_shapes=[
                pltpu.VMEM((BM, HD), jnp.float32),
                pltpu.VMEM((BM, 128), jnp.float32),
                pltpu.VMEM((BM, 128), jnp.float32),
            ],
        ),
        out_shape=jax.ShapeDtypeStruct((T, NH * HD), jnp.float32),
    )(qi_arr, kj_arr, q, k, v)

    h, h2, lg = pl.pallas_call(
        _oproj_kernel,
        grid=(nm,),
        in_specs=[
            pl.BlockSpec((BM, NH * HD), lambda i: (i, 0)),
            pl.BlockSpec((NH * HD, H), lambda i: (0, 0)),
            pl.BlockSpec((BM, H), lambda i: (i, 0)),
            pl.BlockSpec((1, H), lambda i: (0, 0)),
            pl.BlockSpec((H, E), lambda i: (0, 0)),
        ],
        out_specs=[
            pl.BlockSpec((BM, H), lambda i: (i, 0)),
            pl.BlockSpec((BM, H), lambda i: (i, 0)),
            pl.BlockSpec((BM, E), lambda i: (i, 0)),
        ],
        out_shape=[
            jax.ShapeDtypeStruct((T, H), jnp.float32),
            jax.ShapeDtypeStruct((T, H), jnp.float32),
            jax.ShapeDtypeStruct((T, E), jnp.float32),
        ],
    )(o, w_o, hidden_states, ln2, w_router)

    (p1, p2, w1, w2, mt, gid, lo, hi) = pl.pallas_call(
        _router_kernel,
        grid=(1,),
        in_specs=[
            pl.BlockSpec((T, E), lambda i: (0, 0)),
        ],
        out_specs=[
            pl.BlockSpec((T, 1), lambda i: (0, 0)),
            pl.BlockSpec((T, 1), lambda i: (0, 0)),
            pl.BlockSpec((T, 1), lambda i: (0, 0)),
            pl.BlockSpec((T, 1), lambda i: (0, 0)),
            pl.BlockSpec((1, G), lambda i: (0, 0)),
            pl.BlockSpec((1, G), lambda i: (0, 0)),
            pl.BlockSpec((1, G), lambda i: (0, 0)),
            pl.BlockSpec((1, G), lambda i: (0, 0)),
        ],
        out_shape=[
            jax.ShapeDtypeStruct((T, 1), jnp.int32),
            jax.ShapeDtypeStruct((T, 1), jnp.int32),
            jax.ShapeDtypeStruct((T, 1), jnp.float32),
            jax.ShapeDtypeStruct((T, 1), jnp.float32),
            jax.ShapeDtypeStruct((1, G), jnp.int32),
            jax.ShapeDtypeStruct((1, G), jnp.int32),
            jax.ShapeDtypeStruct((1, G), jnp.int32),
            jax.ShapeDtypeStruct((1, G), jnp.int32),
        ],
    )(lg)

    p1f = p1.reshape(T)
    p2f = p2.reshape(T)

    xs = _sc_dispatch(h2, p1f, p2f)

    ys = pl.pallas_call(
        _gmm_kernel,
        grid_spec=pltpu.PrefetchScalarGridSpec(
            num_scalar_prefetch=4,
            grid=(G,),
            in_specs=[
                pl.BlockSpec((GBM, H), lambda g, mt, gid, lo, hi: (mt[g], 0)),
                pl.BlockSpec((1, H, DFF), lambda g, mt, gid, lo, hi: (gid[g], 0, 0)),
                pl.BlockSpec((1, H, DFF), lambda g, mt, gid, lo, hi: (gid[g], 0, 0)),
                pl.BlockSpec((1, DFF, H), lambda g, mt, gid, lo, hi: (gid[g], 0, 0)),
            ],
            out_specs=pl.BlockSpec((GBM, H), lambda g, mt, gid, lo, hi: (mt[g], 0)),
        ),
        out_shape=jax.ShapeDtypeStruct((S, H), jnp.float32),
    )(mt.reshape(G), gid.reshape(G), lo.reshape(G), hi.reshape(G),
      xs, w_gate, w_up, w_down)

    out = _sc_combine(ys, h, p1f, p2f, w1.reshape(T), w2.reshape(T))
    return out


# speedup vs baseline: 1.4570x; 1.4570x over previous
import functools

import jax
import jax.numpy as jnp
from jax import lax
from jax.experimental import pallas as pl
from jax.experimental.pallas import tpu as pltpu
from jax.experimental.pallas import tpu_sc as plsc

T = 2048
H = 2048
NH = 16
NKV = 4
HD = 128
E = 8
TOPK = 2
DFF = 768
EPS = 1e-06
THETA = 10000.0

BM = 256
S = T * TOPK
GBM = 128
GSH = 7
NT = S // GBM
G = NT + E - 1
PREC = jax.lax.Precision.DEFAULT


def _rms_rows(x, w):
    return x * jax.lax.rsqrt(jnp.mean(x * x, axis=-1, keepdims=True) + EPS) * w


def _qkv_kernel(x_ref, ln1_ref, w_ref, qn_ref, kn_ref, pos_ref,
                q_ref, k_ref, v_ref):
    x = x_ref[...]
    h = _rms_rows(x, ln1_ref[...])
    qkv = jnp.dot(h, w_ref[...], preferred_element_type=jnp.float32,
                  precision=PREC)

    pos = pos_ref[:, 0].astype(jnp.float32)
    half = HD // 2
    inv = 1.0 / (THETA ** (jnp.arange(0, half, dtype=jnp.int32).astype(jnp.float32) / half))
    f = pos[:, None] * inv[None, :]
    cos = jnp.cos(f)
    sin = jnp.sin(f)

    def norm_rope(s, nw):
        s = _rms_rows(s, nw)
        x1 = s[:, :half]
        x2 = s[:, half:]
        return jnp.concatenate([x1 * cos - x2 * sin, x2 * cos + x1 * sin],
                               axis=-1)

    qn = qn_ref[...]
    kn = kn_ref[...]
    for hidx in range(NH):
        sl = slice(hidx * HD, (hidx + 1) * HD)
        q_ref[:, sl] = norm_rope(qkv[:, sl], qn)
    for hidx in range(NKV):
        sl = slice(NH * HD + hidx * HD, NH * HD + (hidx + 1) * HD)
        osl = slice(hidx * HD, (hidx + 1) * HD)
        k_ref[:, osl] = norm_rope(qkv[:, sl], kn)
        vsl = slice((NH + NKV) * HD + hidx * HD, (NH + NKV) * HD + (hidx + 1) * HD)
        v_ref[:, osl] = qkv[:, vsl]


def _attn_kernel(q_ref, k_ref, v_ref, o_ref):
    qi = pl.program_id(1)
    q = q_ref[...]
    k = k_ref[...]
    v = v_ref[...]
    s = jax.lax.dot_general(q, k, (((1,), (1,)), ((), ())),
                            preferred_element_type=jnp.float32,
                            precision=PREC) * (HD ** -0.5)
    row = qi * BM + jax.lax.broadcasted_iota(jnp.int32, s.shape, 0)
    col = jax.lax.broadcasted_iota(jnp.int32, s.shape, 1)
    s = jnp.where(col <= row, s, -1e30)
    m = jnp.max(s, axis=-1, keepdims=True)
    p = jnp.exp(s - m)
    l = jnp.sum(p, axis=-1, keepdims=True)
    o = jnp.dot(p, v, preferred_element_type=jnp.float32, precision=PREC)
    o_ref[...] = o / l


def _oproj_kernel(o_ref, w_ref, res_ref, ln2_ref, wr_ref,
                  h_ref, h2_ref, lg_ref):
    h = res_ref[...] + jnp.dot(
        o_ref[...], w_ref[...], preferred_element_type=jnp.float32,
        precision=PREC)
    h_ref[...] = h
    h2 = _rms_rows(h, ln2_ref[...])
    h2_ref[...] = h2
    lg_ref[...] = jnp.dot(h2, wr_ref[...], preferred_element_type=jnp.float32,
                          precision=PREC)


def _cumsum_rows(x):
    s = x
    shift = 1
    while shift < x.shape[0]:
        z = jnp.zeros((shift, x.shape[1]), x.dtype)
        s = s + jnp.concatenate([z, s[:-shift]], axis=0)
        shift *= 2
    return s


def _cumsum_lanes_excl(x):
    n = x.shape[1]
    s = jnp.zeros_like(x)
    acc = x
    shift = 1
    while shift < n:
        z = jnp.zeros((x.shape[0], shift), x.dtype)
        acc = acc + jnp.concatenate([z, acc[:, :-shift]], axis=1)
        shift *= 2
    s = acc - x
    return s


def _router_kernel(lg_ref,
                   p1_ref, p2_ref, w1_ref, w2_ref,
                   mt_ref, gid_ref, lo_ref, hi_ref):
    logits = lg_ref[...]
    mx = jnp.max(logits, axis=-1, keepdims=True)
    ex = jnp.exp(logits - mx)
    p = ex / jnp.sum(ex, axis=-1, keepdims=True)
    lane = jax.lax.broadcasted_iota(jnp.int32, p.shape, 1)
    m1 = jnp.max(p, axis=-1, keepdims=True)
    i1 = jnp.min(jnp.where(p == m1, lane, E), axis=-1, keepdims=True)
    oh1 = lane == i1
    pm = jnp.where(oh1, -jnp.inf, p)
    m2 = jnp.max(pm, axis=-1, keepdims=True)
    i2 = jnp.min(jnp.where(pm == m2, lane, E), axis=-1, keepdims=True)
    oh2 = lane == i2
    tot = m1 + m2
    w1_ref[...] = m1 / tot
    w2_ref[...] = m2 / tot

    cnt = oh1.astype(jnp.int32) + oh2.astype(jnp.int32)
    incl = _cumsum_rows(cnt)
    excl = incl - cnt
    sizes = incl[T - 1:T, :]
    offs = _cumsum_lanes_excl(sizes)
    pos = offs + excl
    p1_ref[...] = jnp.sum(jnp.where(oh1, pos, 0), axis=-1, keepdims=True)
    p2_ref[...] = jnp.sum(jnp.where(oh2, pos, 0), axis=-1, keepdims=True)

    off_lo = offs
    off_hi = offs + sizes
    start_t = off_lo >> GSH
    end_t = (off_hi - 1) >> GSH
    nt = jnp.where(sizes > 0, end_t - start_t + 1, 0)
    estart = _cumsum_lanes_excl(nt)
    n_entries = estart[:, E - 1:E] + nt[:, E - 1:E]

    gi = jax.lax.broadcasted_iota(jnp.int32, (E, G), 1)
    erow = jax.lax.broadcasted_iota(jnp.int32, (E, G), 0)
    est_b = jnp.broadcast_to(estart.reshape(E, 1), (E, G))
    nt_b = jnp.broadcast_to(nt.reshape(E, 1), (E, G))
    st_b = jnp.broadcast_to(start_t.reshape(E, 1), (E, G))
    lo_b = jnp.broadcast_to(off_lo.reshape(E, 1), (E, G))
    hi_b = jnp.broadcast_to(off_hi.reshape(E, 1), (E, G))
    ind = (gi >= est_b) & (gi < est_b + nt_b)
    iz = jnp.zeros((E, G), jnp.int32)
    mt = jnp.sum(jnp.where(ind, st_b + gi - est_b, iz), axis=0, keepdims=True)
    gid = jnp.sum(jnp.where(ind, erow, iz), axis=0, keepdims=True)
    elo = jnp.sum(jnp.where(ind, lo_b, iz), axis=0, keepdims=True)
    ehi = jnp.sum(jnp.where(ind, hi_b, iz), axis=0, keepdims=True)
    tile_lo = mt << GSH
    tile_hi = tile_lo + GBM
    lo = jnp.maximum(elo, tile_lo)
    hi = jnp.minimum(ehi, tile_hi)
    gpad = jax.lax.broadcasted_iota(jnp.int32, (1, G), 1) >= n_entries
    mt_ref[...] = jnp.where(gpad, NT - 1, mt)
    gid_ref[...] = jnp.where(gpad, 0, gid)
    lo_ref[...] = jnp.where(gpad, 0, lo)
    hi_ref[...] = jnp.where(gpad, 0, hi)


def _gmm_kernel(mt_ref, gid_ref, lo_ref, hi_ref,
                x_ref, wg_ref, wu_ref, wd_ref, y_ref):
    g = pl.program_id(0)
    x = x_ref[...]
    gt = jnp.dot(x, wg_ref[0], preferred_element_type=jnp.float32,
                 precision=PREC)
    up = jnp.dot(x, wu_ref[0], preferred_element_type=jnp.float32,
                 precision=PREC)
    a = (gt * jax.lax.logistic(gt)) * up
    y = jnp.dot(a, wd_ref[0], preferred_element_type=jnp.float32,
                precision=PREC)
    rows = mt_ref[g] * GBM + jax.lax.broadcasted_iota(
        jnp.int32, (GBM, 1), 0)
    mask = (rows >= lo_ref[g]) & (rows < hi_ref[g])
    contrib = jnp.where(mask, y, 0.0)

    prev = jnp.where(g == 0, -1, mt_ref[jnp.maximum(g - 1, 0)])
    first = mt_ref[g] != prev

    @pl.when(first)
    def _():
        y_ref[...] = contrib

    @pl.when(jnp.logical_not(first))
    def _():
        y_ref[...] += contrib


NWORK = 32
TPW = T // NWORK
CH = 16


@functools.cache
def _build_sc_dispatch():
    mesh = plsc.VectorSubcoreMesh(core_axis_name="c", subcore_axis_name="s")

    @functools.partial(
        pl.kernel, mesh=mesh,
        out_type=jax.ShapeDtypeStruct((S, H), jnp.float32),
        scratch_types=[
            pltpu.VMEM((CH,), jnp.int32),
            pltpu.VMEM((CH,), jnp.int32),
            pltpu.VMEM((CH, H), jnp.float32),
            pltpu.SemaphoreType.DMA,
            pltpu.SemaphoreType.DMA,
        ],
    )
    def dispatch(h2_hbm, p1_hbm, p2_hbm, x_hbm, idx1_v, idx2_v, rows_v,
                 sem, sem2):
        wid = lax.axis_index("s") * 2 + lax.axis_index("c")
        for j in range(TPW // CH):
            base = wid * TPW + j * CH
            pltpu.sync_copy(h2_hbm.at[pl.ds(base, CH)], rows_v)
            pltpu.sync_copy(p1_hbm.at[pl.ds(base, CH)], idx1_v)
            pltpu.sync_copy(p2_hbm.at[pl.ds(base, CH)], idx2_v)
            c1 = pltpu.async_copy(rows_v, x_hbm.at[idx1_v], sem)
            c2 = pltpu.async_copy(rows_v, x_hbm.at[idx2_v], sem2)
            c1.wait()
            c2.wait()

    return dispatch


@functools.cache
def _build_sc_combine():
    mesh = plsc.VectorSubcoreMesh(core_axis_name="c", subcore_axis_name="s")

    @functools.partial(
        pl.kernel, mesh=mesh,
        out_type=jax.ShapeDtypeStruct((T, H), jnp.float32),
        scratch_types=[
            pltpu.VMEM((CH,), jnp.int32),
            pltpu.VMEM((CH,), jnp.int32),
            pltpu.VMEM((CH,), jnp.float32),
            pltpu.VMEM((CH,), jnp.float32),
            pltpu.VMEM((CH, H), jnp.float32),
            pltpu.VMEM((CH, H), jnp.float32),
            pltpu.VMEM((CH, H), jnp.float32),
            pltpu.SemaphoreType.DMA,
            pltpu.SemaphoreType.DMA,
        ],
    )
    def combine(ys_hbm, res_hbm, p1_hbm, p2_hbm, w1_hbm, w2_hbm, out_hbm,
                idx1_v, idx2_v, w1_v, w2_v, y1_v, y2_v, acc_v, sem, sem2):
        wid = lax.axis_index("s") * 2 + lax.axis_index("c")
        for j in range(TPW // CH):
            base = wid * TPW + j * CH
            pltpu.sync_copy(p1_hbm.at[pl.ds(base, CH)], idx1_v)
            pltpu.sync_copy(p2_hbm.at[pl.ds(base, CH)], idx2_v)
            c1 = pltpu.async_copy(ys_hbm.at[idx1_v], y1_v, sem)
            c2 = pltpu.async_copy(ys_hbm.at[idx2_v], y2_v, sem2)
            pltpu.sync_copy(w1_hbm.at[pl.ds(base, CH)], w1_v)
            pltpu.sync_copy(w2_hbm.at[pl.ds(base, CH)], w2_v)
            pltpu.sync_copy(res_hbm.at[pl.ds(base, CH)], acc_v)
            c1.wait()
            c2.wait()
            w1x = w1_v[...]
            w2x = w2_v[...]
            for t in range(CH):
                tvec = jnp.full((16,), t, jnp.int32)
                w1b = w1x.at[tvec].get(mode="promise_in_bounds")
                w2b = w2x.at[tvec].get(mode="promise_in_bounds")

                def body(c, _, t=t, w1b=w1b, w2b=w2b):
                    sl = pl.ds(c * 16, 16)
                    acc_v[t, sl] = (acc_v[t, sl] + w1b * y1_v[t, sl]
                                    + w2b * y2_v[t, sl])
                    return 0

                lax.fori_loop(0, H // 16, body, 0, unroll=8)
            pltpu.sync_copy(acc_v, out_hbm.at[pl.ds(base, CH)])

    return combine


def _sc_dispatch(h2, p1f, p2f):
    return _build_sc_dispatch()(h2, p1f, p2f)


def _sc_combine(ys, res, p1f, p2f, w1f, w2f):
    return _build_sc_combine()(ys, res, p1f, p2f, w1f, w2f)


@jax.jit
def kernel(hidden_states, positions, ln1_w, w_qkv, q_norm_w, k_norm_w, w_o,
           ln2_w, w_router, w_gate, w_up, w_down):
    posc = positions.reshape(T, 1)
    ln1 = ln1_w.reshape(1, H)
    ln2 = ln2_w.reshape(1, H)
    qn = q_norm_w.reshape(1, HD)
    kn = k_norm_w.reshape(1, HD)
    nm = T // BM

    q, k, v = pl.pallas_call(
        _qkv_kernel,
        grid=(nm,),
        in_specs=[
            pl.BlockSpec((BM, H), lambda i: (i, 0)),
            pl.BlockSpec((1, H), lambda i: (0, 0)),
            pl.BlockSpec((H, (NH + 2 * NKV) * HD), lambda i: (0, 0)),
            pl.BlockSpec((1, HD), lambda i: (0, 0)),
            pl.BlockSpec((1, HD), lambda i: (0, 0)),
            pl.BlockSpec((BM, 1), lambda i: (i, 0)),
        ],
        out_specs=[
            pl.BlockSpec((BM, NH * HD), lambda i: (i, 0)),
            pl.BlockSpec((BM, NKV * HD), lambda i: (i, 0)),
            pl.BlockSpec((BM, NKV * HD), lambda i: (i, 0)),
        ],
        out_shape=[
            jax.ShapeDtypeStruct((T, NH * HD), jnp.float32),
            jax.ShapeDtypeStruct((T, NKV * HD), jnp.float32),
            jax.ShapeDtypeStruct((T, NKV * HD), jnp.float32),
        ],
    )(hidden_states, ln1, w_qkv, qn, kn, posc)

    rep = NH // NKV
    o = pl.pallas_call(
        _attn_kernel,
        grid=(NH, nm),
        in_specs=[
            pl.BlockSpec((BM, HD), lambda h, i: (i, h)),
            pl.BlockSpec((T, HD), lambda h, i: (0, h // rep)),
            pl.BlockSpec((T, HD), lambda h, i: (0, h // rep)),
        ],
        out_specs=pl.BlockSpec((BM, HD), lambda h, i: (i, h)),
        out_shape=jax.ShapeDtypeStruct((T, NH * HD), jnp.float32),
    )(q, k, v)

    h, h2, lg = pl.pallas_call(
        _oproj_kernel,
        grid=(nm,),
        in_specs=[
            pl.BlockSpec((BM, NH * HD), lambda i: (i, 0)),
            pl.BlockSpec((NH * HD, H), lambda i: (0, 0)),
            pl.BlockSpec((BM, H), lambda i: (i, 0)),
            pl.BlockSpec((1, H), lambda i: (0, 0)),
            pl.BlockSpec((H, E), lambda i: (0, 0)),
        ],
        out_specs=[
            pl.BlockSpec((BM, H), lambda i: (i, 0)),
            pl.BlockSpec((BM, H), lambda i: (i, 0)),
            pl.BlockSpec((BM, E), lambda i: (i, 0)),
        ],
        out_shape=[
            jax.ShapeDtypeStruct((T, H), jnp.float32),
            jax.ShapeDtypeStruct((T, H), jnp.float32),
            jax.ShapeDtypeStruct((T, E), jnp.float32),
        ],
    )(o, w_o, hidden_states, ln2, w_router)

    (p1, p2, w1, w2, mt, gid, lo, hi) = pl.pallas_call(
        _router_kernel,
        grid=(1,),
        in_specs=[
            pl.BlockSpec((T, E), lambda i: (0, 0)),
        ],
        out_specs=[
            pl.BlockSpec((T, 1), lambda i: (0, 0)),
            pl.BlockSpec((T, 1), lambda i: (0, 0)),
            pl.BlockSpec((T, 1), lambda i: (0, 0)),
            pl.BlockSpec((T, 1), lambda i: (0, 0)),
            pl.BlockSpec((1, G), lambda i: (0, 0)),
            pl.BlockSpec((1, G), lambda i: (0, 0)),
            pl.BlockSpec((1, G), lambda i: (0, 0)),
            pl.BlockSpec((1, G), lambda i: (0, 0)),
        ],
        out_shape=[
            jax.ShapeDtypeStruct((T, 1), jnp.int32),
            jax.ShapeDtypeStruct((T, 1), jnp.int32),
            jax.ShapeDtypeStruct((T, 1), jnp.float32),
            jax.ShapeDtypeStruct((T, 1), jnp.float32),
            jax.ShapeDtypeStruct((1, G), jnp.int32),
            jax.ShapeDtypeStruct((1, G), jnp.int32),
            jax.ShapeDtypeStruct((1, G), jnp.int32),
            jax.ShapeDtypeStruct((1, G), jnp.int32),
        ],
    )(lg)

    p1f = p1.reshape(T)
    p2f = p2.reshape(T)

    xs = _sc_dispatch(h2, p1f, p2f)

    ys = pl.pallas_call(
        _gmm_kernel,
        grid_spec=pltpu.PrefetchScalarGridSpec(
            num_scalar_prefetch=4,
            grid=(G,),
            in_specs=[
                pl.BlockSpec((GBM, H), lambda g, mt, gid, lo, hi: (mt[g], 0)),
                pl.BlockSpec((1, H, DFF), lambda g, mt, gid, lo, hi: (gid[g], 0, 0)),
                pl.BlockSpec((1, H, DFF), lambda g, mt, gid, lo, hi: (gid[g], 0, 0)),
                pl.BlockSpec((1, DFF, H), lambda g, mt, gid, lo, hi: (gid[g], 0, 0)),
            ],
            out_specs=pl.BlockSpec((GBM, H), lambda g, mt, gid, lo, hi: (mt[g], 0)),
        ),
        out_shape=jax.ShapeDtypeStruct((S, H), jnp.float32),
    )(mt.reshape(G), gid.reshape(G), lo.reshape(G), hi.reshape(G),
      xs, w_gate, w_up, w_down)

    out = _sc_combine(ys, h, p1f, p2f, w1.reshape(T), w2.reshape(T))
    return out


# speedup vs baseline: 1.5078x; 1.0348x over previous
import functools

import jax
import jax.numpy as jnp
from jax import lax
from jax.experimental import pallas as pl
from jax.experimental.pallas import tpu as pltpu
from jax.experimental.pallas import tpu_sc as plsc

T = 2048
H = 2048
NH = 16
NKV = 4
HD = 128
E = 8
TOPK = 2
DFF = 768
EPS = 1e-06
THETA = 10000.0

BM = 256
S = T * TOPK
GBM = 128
GSH = 7
NT = S // GBM
G = NT + E - 1
PREC = jax.lax.Precision.DEFAULT


def _rms_rows(x, w):
    return x * jax.lax.rsqrt(jnp.mean(x * x, axis=-1, keepdims=True) + EPS) * w


def _qkv_kernel(x_ref, ln1_ref, w_ref, qn_ref, kn_ref, pos_ref,
                q_ref, k_ref, v_ref):
    x = x_ref[...]
    h = _rms_rows(x, ln1_ref[...])
    qkv = jnp.dot(h, w_ref[...], preferred_element_type=jnp.float32,
                  precision=PREC)

    pos = pos_ref[:, 0].astype(jnp.float32)
    half = HD // 2
    inv = 1.0 / (THETA ** (jnp.arange(0, half, dtype=jnp.int32).astype(jnp.float32) / half))
    f = pos[:, None] * inv[None, :]
    cos = jnp.cos(f)
    sin = jnp.sin(f)

    def norm_rope(s, nw):
        s = _rms_rows(s, nw)
        x1 = s[:, :half]
        x2 = s[:, half:]
        return jnp.concatenate([x1 * cos - x2 * sin, x2 * cos + x1 * sin],
                               axis=-1)

    qn = qn_ref[...]
    kn = kn_ref[...]
    for hidx in range(NH):
        sl = slice(hidx * HD, (hidx + 1) * HD)
        q_ref[:, sl] = norm_rope(qkv[:, sl], qn)
    for hidx in range(NKV):
        sl = slice(NH * HD + hidx * HD, NH * HD + (hidx + 1) * HD)
        osl = slice(hidx * HD, (hidx + 1) * HD)
        k_ref[:, osl] = norm_rope(qkv[:, sl], kn)
        vsl = slice((NH + NKV) * HD + hidx * HD, (NH + NKV) * HD + (hidx + 1) * HD)
        v_ref[:, osl] = qkv[:, vsl]


def _attn_kernel(q_ref, k_ref, v_ref, o_ref):
    qi = pl.program_id(1)
    q = q_ref[...]
    k = k_ref[...]
    v = v_ref[...]
    s = jax.lax.dot_general(q, k, (((1,), (1,)), ((), ())),
                            preferred_element_type=jnp.float32,
                            precision=PREC) * (HD ** -0.5)
    row = qi * BM + jax.lax.broadcasted_iota(jnp.int32, s.shape, 0)
    col = jax.lax.broadcasted_iota(jnp.int32, s.shape, 1)
    s = jnp.where(col <= row, s, -1e30)
    m = jnp.max(s, axis=-1, keepdims=True)
    p = jnp.exp(s - m)
    l = jnp.sum(p, axis=-1, keepdims=True)
    o = jnp.dot(p, v, preferred_element_type=jnp.float32, precision=PREC)
    o_ref[...] = o / l


def _oproj_kernel(o_ref, w_ref, res_ref, ln2_ref, wr_ref,
                  h_ref, h2_ref, lg_ref):
    h = res_ref[...] + jnp.dot(
        o_ref[...], w_ref[...], preferred_element_type=jnp.float32,
        precision=PREC)
    h_ref[...] = h
    h2 = _rms_rows(h, ln2_ref[...])
    h2_ref[...] = h2
    lg_ref[...] = jnp.dot(h2, wr_ref[...], preferred_element_type=jnp.float32,
                          precision=PREC)


def _cumsum_rows(x):
    s = x
    shift = 1
    while shift < x.shape[0]:
        z = jnp.zeros((shift, x.shape[1]), x.dtype)
        s = s + jnp.concatenate([z, s[:-shift]], axis=0)
        shift *= 2
    return s


def _cumsum_lanes_excl(x):
    n = x.shape[1]
    s = jnp.zeros_like(x)
    acc = x
    shift = 1
    while shift < n:
        z = jnp.zeros((x.shape[0], shift), x.dtype)
        acc = acc + jnp.concatenate([z, acc[:, :-shift]], axis=1)
        shift *= 2
    s = acc - x
    return s


def _router_kernel(lg_ref,
                   p1_ref, p2_ref, w1_ref, w2_ref,
                   mt_ref, gid_ref, lo_ref, hi_ref):
    logits = lg_ref[...]
    mx = jnp.max(logits, axis=-1, keepdims=True)
    ex = jnp.exp(logits - mx)
    p = ex / jnp.sum(ex, axis=-1, keepdims=True)
    lane = jax.lax.broadcasted_iota(jnp.int32, p.shape, 1)
    m1 = jnp.max(p, axis=-1, keepdims=True)
    i1 = jnp.min(jnp.where(p == m1, lane, E), axis=-1, keepdims=True)
    oh1 = lane == i1
    pm = jnp.where(oh1, -jnp.inf, p)
    m2 = jnp.max(pm, axis=-1, keepdims=True)
    i2 = jnp.min(jnp.where(pm == m2, lane, E), axis=-1, keepdims=True)
    oh2 = lane == i2
    tot = m1 + m2
    w1_ref[...] = m1 / tot
    w2_ref[...] = m2 / tot

    cnt = oh1.astype(jnp.int32) + oh2.astype(jnp.int32)
    incl = _cumsum_rows(cnt)
    excl = incl - cnt
    sizes = incl[T - 1:T, :]
    offs = _cumsum_lanes_excl(sizes)
    pos = offs + excl
    p1_ref[...] = jnp.sum(jnp.where(oh1, pos, 0), axis=-1, keepdims=True)
    p2_ref[...] = jnp.sum(jnp.where(oh2, pos, 0), axis=-1, keepdims=True)

    off_lo = offs
    off_hi = offs + sizes
    start_t = off_lo >> GSH
    end_t = (off_hi - 1) >> GSH
    nt = jnp.where(sizes > 0, end_t - start_t + 1, 0)
    estart = _cumsum_lanes_excl(nt)
    n_entries = estart[:, E - 1:E] + nt[:, E - 1:E]

    gi = jax.lax.broadcasted_iota(jnp.int32, (E, G), 1)
    erow = jax.lax.broadcasted_iota(jnp.int32, (E, G), 0)
    est_b = jnp.broadcast_to(estart.reshape(E, 1), (E, G))
    nt_b = jnp.broadcast_to(nt.reshape(E, 1), (E, G))
    st_b = jnp.broadcast_to(start_t.reshape(E, 1), (E, G))
    lo_b = jnp.broadcast_to(off_lo.reshape(E, 1), (E, G))
    hi_b = jnp.broadcast_to(off_hi.reshape(E, 1), (E, G))
    ind = (gi >= est_b) & (gi < est_b + nt_b)
    iz = jnp.zeros((E, G), jnp.int32)
    mt = jnp.sum(jnp.where(ind, st_b + gi - est_b, iz), axis=0, keepdims=True)
    gid = jnp.sum(jnp.where(ind, erow, iz), axis=0, keepdims=True)
    elo = jnp.sum(jnp.where(ind, lo_b, iz), axis=0, keepdims=True)
    ehi = jnp.sum(jnp.where(ind, hi_b, iz), axis=0, keepdims=True)
    tile_lo = mt << GSH
    tile_hi = tile_lo + GBM
    lo = jnp.maximum(elo, tile_lo)
    hi = jnp.minimum(ehi, tile_hi)
    gpad = jax.lax.broadcasted_iota(jnp.int32, (1, G), 1) >= n_entries
    mt_ref[...] = jnp.where(gpad, NT - 1, mt)
    gid_ref[...] = jnp.where(gpad, 0, gid)
    lo_ref[...] = jnp.where(gpad, 0, lo)
    hi_ref[...] = jnp.where(gpad, 0, hi)


def _gmm_kernel(mt_ref, gid_ref, lo_ref, hi_ref,
                x_ref, wg_ref, wu_ref, wd_ref, y_ref):
    g = pl.program_id(0)
    x = x_ref[...]
    gt = jnp.dot(x, wg_ref[0], preferred_element_type=jnp.float32,
                 precision=PREC)
    up = jnp.dot(x, wu_ref[0], preferred_element_type=jnp.float32,
                 precision=PREC)
    a = (gt * jax.lax.logistic(gt)) * up
    y = jnp.dot(a, wd_ref[0], preferred_element_type=jnp.float32,
                precision=PREC)
    rows = mt_ref[g] * GBM + jax.lax.broadcasted_iota(
        jnp.int32, (GBM, 1), 0)
    mask = (rows >= lo_ref[g]) & (rows < hi_ref[g])
    contrib = jnp.where(mask, y, 0.0)

    prev = jnp.where(g == 0, -1, mt_ref[jnp.maximum(g - 1, 0)])
    first = mt_ref[g] != prev

    @pl.when(first)
    def _():
        y_ref[...] = contrib

    @pl.when(jnp.logical_not(first))
    def _():
        y_ref[...] += contrib


NWORK = 32
TPW = T // NWORK
CH = 16


@functools.cache
def _build_sc_dispatch():
    mesh = plsc.VectorSubcoreMesh(core_axis_name="c", subcore_axis_name="s")

    @functools.partial(
        pl.kernel, mesh=mesh,
        out_type=jax.ShapeDtypeStruct((S, H), jnp.float32),
        scratch_types=[
            pltpu.VMEM((CH,), jnp.int32),
            pltpu.VMEM((CH,), jnp.int32),
            pltpu.VMEM((CH, H), jnp.float32),
            pltpu.SemaphoreType.DMA,
            pltpu.SemaphoreType.DMA,
        ],
    )
    def dispatch(h2_hbm, p1_hbm, p2_hbm, x_hbm, idx1_v, idx2_v, rows_v,
                 sem, sem2):
        wid = lax.axis_index("s") * 2 + lax.axis_index("c")
        for j in range(TPW // CH):
            base = wid * TPW + j * CH
            pltpu.sync_copy(h2_hbm.at[pl.ds(base, CH)], rows_v)
            pltpu.sync_copy(p1_hbm.at[pl.ds(base, CH)], idx1_v)
            pltpu.sync_copy(p2_hbm.at[pl.ds(base, CH)], idx2_v)
            c1 = pltpu.async_copy(rows_v, x_hbm.at[idx1_v], sem)
            c2 = pltpu.async_copy(rows_v, x_hbm.at[idx2_v], sem2)
            c1.wait()
            c2.wait()

    return dispatch


@functools.cache
def _build_sc_gather2():
    mesh = plsc.VectorSubcoreMesh(core_axis_name="c", subcore_axis_name="s")

    @functools.partial(
        pl.kernel, mesh=mesh,
        out_type=[
            jax.ShapeDtypeStruct((T, H), jnp.float32),
            jax.ShapeDtypeStruct((T, H), jnp.float32),
        ],
        scratch_types=[
            pltpu.VMEM((CH,), jnp.int32),
            pltpu.VMEM((CH,), jnp.int32),
            pltpu.VMEM((CH, H), jnp.float32),
            pltpu.VMEM((CH, H), jnp.float32),
            pltpu.SemaphoreType.DMA,
            pltpu.SemaphoreType.DMA,
        ],
    )
    def gather2(ys_hbm, p1_hbm, p2_hbm, y1_hbm, y2_hbm,
                idx1_v, idx2_v, y1_v, y2_v, sem, sem2):
        wid = lax.axis_index("s") * 2 + lax.axis_index("c")
        for j in range(TPW // CH):
            base = wid * TPW + j * CH
            pltpu.sync_copy(p1_hbm.at[pl.ds(base, CH)], idx1_v)
            pltpu.sync_copy(p2_hbm.at[pl.ds(base, CH)], idx2_v)
            c1 = pltpu.async_copy(ys_hbm.at[idx1_v], y1_v, sem)
            c2 = pltpu.async_copy(ys_hbm.at[idx2_v], y2_v, sem2)
            c1.wait()
            c2.wait()
            pltpu.sync_copy(y1_v, y1_hbm.at[pl.ds(base, CH)])
            pltpu.sync_copy(y2_v, y2_hbm.at[pl.ds(base, CH)])

    return gather2


def _merge_kernel(res_ref, w1_ref, w2_ref, y1_ref, y2_ref, out_ref):
    out_ref[...] = (res_ref[...] + w1_ref[...] * y1_ref[...]
                    + w2_ref[...] * y2_ref[...])


def _sc_dispatch(h2, p1f, p2f):
    return _build_sc_dispatch()(h2, p1f, p2f)


def _sc_combine(ys, res, p1f, p2f, w1f, w2f):
    y1g, y2g = _build_sc_gather2()(ys, p1f, p2f)
    nm = T // BM
    return pl.pallas_call(
        _merge_kernel,
        grid=(nm,),
        in_specs=[
            pl.BlockSpec((BM, H), lambda i: (i, 0)),
            pl.BlockSpec((BM, 1), lambda i: (i, 0)),
            pl.BlockSpec((BM, 1), lambda i: (i, 0)),
            pl.BlockSpec((BM, H), lambda i: (i, 0)),
            pl.BlockSpec((BM, H), lambda i: (i, 0)),
        ],
        out_specs=pl.BlockSpec((BM, H), lambda i: (i, 0)),
        out_shape=jax.ShapeDtypeStruct((T, H), jnp.float32),
    )(res, w1f.reshape(T, 1), w2f.reshape(T, 1), y1g, y2g)


@jax.jit
def kernel(hidden_states, positions, ln1_w, w_qkv, q_norm_w, k_norm_w, w_o,
           ln2_w, w_router, w_gate, w_up, w_down):
    posc = positions.reshape(T, 1)
    ln1 = ln1_w.reshape(1, H)
    ln2 = ln2_w.reshape(1, H)
    qn = q_norm_w.reshape(1, HD)
    kn = k_norm_w.reshape(1, HD)
    nm = T // BM

    q, k, v = pl.pallas_call(
        _qkv_kernel,
        grid=(nm,),
        in_specs=[
            pl.BlockSpec((BM, H), lambda i: (i, 0)),
            pl.BlockSpec((1, H), lambda i: (0, 0)),
            pl.BlockSpec((H, (NH + 2 * NKV) * HD), lambda i: (0, 0)),
            pl.BlockSpec((1, HD), lambda i: (0, 0)),
            pl.BlockSpec((1, HD), lambda i: (0, 0)),
            pl.BlockSpec((BM, 1), lambda i: (i, 0)),
        ],
        out_specs=[
            pl.BlockSpec((BM, NH * HD), lambda i: (i, 0)),
            pl.BlockSpec((BM, NKV * HD), lambda i: (i, 0)),
            pl.BlockSpec((BM, NKV * HD), lambda i: (i, 0)),
        ],
        out_shape=[
            jax.ShapeDtypeStruct((T, NH * HD), jnp.float32),
            jax.ShapeDtypeStruct((T, NKV * HD), jnp.float32),
            jax.ShapeDtypeStruct((T, NKV * HD), jnp.float32),
        ],
    )(hidden_states, ln1, w_qkv, qn, kn, posc)

    rep = NH // NKV
    o = pl.pallas_call(
        _attn_kernel,
        grid=(NH, nm),
        in_specs=[
            pl.BlockSpec((BM, HD), lambda h, i: (i, h)),
            pl.BlockSpec((T, HD), lambda h, i: (0, h // rep)),
            pl.BlockSpec((T, HD), lambda h, i: (0, h // rep)),
        ],
        out_specs=pl.BlockSpec((BM, HD), lambda h, i: (i, h)),
        out_shape=jax.ShapeDtypeStruct((T, NH * HD), jnp.float32),
    )(q, k, v)

    h, h2, lg = pl.pallas_call(
        _oproj_kernel,
        grid=(nm,),
        in_specs=[
            pl.BlockSpec((BM, NH * HD), lambda i: (i, 0)),
            pl.BlockSpec((NH * HD, H), lambda i: (0, 0)),
            pl.BlockSpec((BM, H), lambda i: (i, 0)),
            pl.BlockSpec((1, H), lambda i: (0, 0)),
            pl.BlockSpec((H, E), lambda i: (0, 0)),
        ],
        out_specs=[
            pl.BlockSpec((BM, H), lambda i: (i, 0)),
            pl.BlockSpec((BM, H), lambda i: (i, 0)),
            pl.BlockSpec((BM, E), lambda i: (i, 0)),
        ],
        out_shape=[
            jax.ShapeDtypeStruct((T, H), jnp.float32),
            jax.ShapeDtypeStruct((T, H), jnp.float32),
            jax.ShapeDtypeStruct((T, E), jnp.float32),
        ],
    )(o, w_o, hidden_states, ln2, w_router)

    (p1, p2, w1, w2, mt, gid, lo, hi) = pl.pallas_call(
        _router_kernel,
        grid=(1,),
        in_specs=[
            pl.BlockSpec((T, E), lambda i: (0, 0)),
        ],
        out_specs=[
            pl.BlockSpec((T, 1), lambda i: (0, 0)),
            pl.BlockSpec((T, 1), lambda i: (0, 0)),
            pl.BlockSpec((T, 1), lambda i: (0, 0)),
            pl.BlockSpec((T, 1), lambda i: (0, 0)),
            pl.BlockSpec((1, G), lambda i: (0, 0)),
            pl.BlockSpec((1, G), lambda i: (0, 0)),
            pl.BlockSpec((1, G), lambda i: (0, 0)),
            pl.BlockSpec((1, G), lambda i: (0, 0)),
        ],
        out_shape=[
            jax.ShapeDtypeStruct((T, 1), jnp.int32),
            jax.ShapeDtypeStruct((T, 1), jnp.int32),
            jax.ShapeDtypeStruct((T, 1), jnp.float32),
            jax.ShapeDtypeStruct((T, 1), jnp.float32),
            jax.ShapeDtypeStruct((1, G), jnp.int32),
            jax.ShapeDtypeStruct((1, G), jnp.int32),
            jax.ShapeDtypeStruct((1, G), jnp.int32),
            jax.ShapeDtypeStruct((1, G), jnp.int32),
        ],
    )(lg)

    p1f = p1.reshape(T)
    p2f = p2.reshape(T)

    xs = _sc_dispatch(h2, p1f, p2f)

    ys = pl.pallas_call(
        _gmm_kernel,
        grid_spec=pltpu.PrefetchScalarGridSpec(
            num_scalar_prefetch=4,
            grid=(G,),
            in_specs=[
                pl.BlockSpec((GBM, H), lambda g, mt, gid, lo, hi: (mt[g], 0)),
                pl.BlockSpec((1, H, DFF), lambda g, mt, gid, lo, hi: (gid[g], 0, 0)),
                pl.BlockSpec((1, H, DFF), lambda g, mt, gid, lo, hi: (gid[g], 0, 0)),
                pl.BlockSpec((1, DFF, H), lambda g, mt, gid, lo, hi: (gid[g], 0, 0)),
            ],
            out_specs=pl.BlockSpec((GBM, H), lambda g, mt, gid, lo, hi: (mt[g], 0)),
        ),
        out_shape=jax.ShapeDtypeStruct((S, H), jnp.float32),
    )(mt.reshape(G), gid.reshape(G), lo.reshape(G), hi.reshape(G),
      xs, w_gate, w_up, w_down)

    out = _sc_combine(ys, h, p1f, p2f, w1.reshape(T), w2.reshape(T))
    return out
